# baseline (speedup 1.0000x reference)
import jax
import jax.numpy as jnp
from jax import lax
from jax.experimental import pallas as pl
from jax.experimental.pallas import tpu as pltpu
from jax.experimental.pallas import tpu_sc as plsc

N = 10000
B = 128
E = 320000
NC, NS, L = 2, 16, 16
W = NC * NS
BLK = 128
BPW = 80
NBLK = BPW * W
E_PAD = NBLK * BLK
STRIPE = 624
STRIPE_LAST = N - STRIPE * (NS - 1)
GRID = 10


def _exp_body(x_ref, o_ref):
    o_ref[...] = jnp.exp(x_ref[...])


def _log_body(p_ref, o_ref):
    s = p_ref[0] + p_ref[1]
    o_ref[...] = jnp.log(jnp.maximum(s, 1e-30))


def _sc_body(ev, meta, pf, zeros, out, meta_v, p_v, rows_v, s_sh,
             gsem, ssem, msem):
    cid = lax.axis_index("c")
    sid = lax.axis_index("s")
    wid = cid * NS + sid

    pltpu.sync_copy(meta.at[wid], meta_v.at[0])
    pltpu.sync_copy(pf.at[wid], p_v.at[0])
    pltpu.async_copy(ev.at[meta_v.at[0, 0]], rows_v.at[0], gsem)
    pltpu.async_copy(meta.at[wid + W], meta_v.at[1], msem)
    pltpu.async_copy(pf.at[wid + W], p_v.at[1], msem)

    r0 = sid * STRIPE

    @pl.when(sid < NS - 1)
    def _():
        pltpu.sync_copy(zeros.at[pl.ds(r0, STRIPE)],
                        s_sh.at[pl.ds(r0, STRIPE)])

    @pl.when(sid == NS - 1)
    def _():
        pltpu.sync_copy(zeros.at[pl.ds(r0, STRIPE_LAST)],
                        s_sh.at[pl.ds(r0, STRIPE_LAST)])

    plsc.subcore_barrier()

    def outer(t2, carry):
        for u in range(4):
            b = u % 2
            m = u
            m1 = (u + 1) % 4
            m2 = (u + 2) % 4

            pltpu.make_async_copy(ev.at[meta_v.at[m, 0]], rows_v.at[b],
                                  gsem).wait()

            def mul_group(g, c):
                p16 = p_v[m, 0, pl.ds(g * L, L)]
                for k in range(L):
                    ps = jnp.full((L,), p16[k], jnp.float32)
                    row = g * L + k
                    for j in range(B // L):
                        sl = (b, row, pl.ds(j * L, L))
                        rows_v[sl] = rows_v[sl] * ps
                return c

            lax.fori_loop(0, BLK // L, mul_group, 0)

            pltpu.async_copy(rows_v.at[b], s_sh.at[meta_v.at[m, 1]], ssem,
                             add=True)

            def wait_prev_scatter():
                pltpu.make_async_copy(
                    rows_v.at[1 - b], s_sh.at[meta_v.at[(u + 3) % 4, 1]],
                    ssem).wait()

            if u == 0:
                pl.when(t2 >= 1)(wait_prev_scatter)
            else:
                wait_prev_scatter()

            pltpu.make_async_copy(meta.at[wid], meta_v.at[m1], msem).wait()
            pltpu.make_async_copy(pf.at[wid], p_v.at[m1], msem).wait()
            pltpu.async_copy(ev.at[meta_v.at[m1, 0]], rows_v.at[1 - b], gsem)

            k2 = 4 * t2 + u + 2
            k2 = jnp.where(k2 < BPW, k2, 0)
            pltpu.async_copy(meta.at[wid + k2 * W], meta_v.at[m2], msem)
            pltpu.async_copy(pf.at[wid + k2 * W], p_v.at[m2], msem)
        return carry

    lax.fori_loop(0, BPW // 4, outer, 0)

    pltpu.make_async_copy(rows_v.at[1], s_sh.at[meta_v.at[3, 1]], ssem).wait()
    pltpu.make_async_copy(ev.at[meta_v.at[0, 0]], rows_v.at[0], gsem).wait()
    pltpu.make_async_copy(meta.at[wid], meta_v.at[1], msem).wait()
    pltpu.make_async_copy(pf.at[wid], p_v.at[1], msem).wait()

    plsc.subcore_barrier()

    @pl.when(sid < NS - 1)
    def _():
        pltpu.sync_copy(s_sh.at[pl.ds(r0, STRIPE)],
                        out.at[cid, pl.ds(r0, STRIPE)])

    @pl.when(sid == NS - 1)
    def _():
        pltpu.sync_copy(s_sh.at[pl.ds(r0, STRIPE_LAST)],
                        out.at[cid, pl.ds(r0, STRIPE_LAST)])


def kernel(ch_vals, edge_src, edge_dst, params):
    ev = pl.pallas_call(
        _exp_body,
        grid=(GRID,),
        in_specs=[pl.BlockSpec((N // GRID, B), lambda i: (i, 0))],
        out_specs=pl.BlockSpec((N // GRID, B), lambda i: (i, 0)),
        out_shape=jax.ShapeDtypeStruct((N, B), jnp.float32),
    )(ch_vals)

    pad = E_PAD - E
    zpad = jnp.zeros((pad,), jnp.int32)
    src_p = jnp.concatenate([edge_src, zpad]).reshape(NBLK, BLK)
    dst_p = jnp.concatenate([edge_dst, zpad]).reshape(NBLK, BLK)
    meta = jnp.stack([src_p, dst_p], axis=1)
    pf = jnp.concatenate([params, jnp.zeros((pad,), jnp.float32)]
                         ).reshape(NBLK, 1, BLK)
    zeros = jnp.zeros((N, B), jnp.float32)

    sc = pl.kernel(
        _sc_body,
        out_type=jax.ShapeDtypeStruct((NC, N, B), jnp.float32),
        mesh=plsc.VectorSubcoreMesh(core_axis_name="c", subcore_axis_name="s"),
        scratch_types=[
            pltpu.VMEM((4, 2, BLK), jnp.int32),
            pltpu.VMEM((4, 1, BLK), jnp.float32),
            pltpu.VMEM((2, BLK, B), jnp.float32),
            pltpu.VMEM_SHARED((N, B), jnp.float32),
            pltpu.SemaphoreType.DMA,
            pltpu.SemaphoreType.DMA,
            pltpu.SemaphoreType.DMA,
        ],
    )
    partial = sc(ev, meta, pf, zeros)

    out = pl.pallas_call(
        _log_body,
        grid=(GRID,),
        in_specs=[pl.BlockSpec((NC, N // GRID, B), lambda i: (0, i, 0))],
        out_specs=pl.BlockSpec((N // GRID, B), lambda i: (i, 0)),
        out_shape=jax.ShapeDtypeStruct((N, B), jnp.float32),
    )(partial)
    return out


# speedup vs baseline: 1.0717x; 1.0717x over previous
import jax
import jax.numpy as jnp
from jax import lax
from jax.experimental import pallas as pl
from jax.experimental.pallas import tpu as pltpu
from jax.experimental.pallas import tpu_sc as plsc

N = 10000
B = 128
E = 320000
NC, NS, L = 2, 16, 16
W = NC * NS
BLK = 128
BPW = 80
NBLK = BPW * W
E_PAD = NBLK * BLK
STRIPE = 624
STRIPE_LAST = N - STRIPE * (NS - 1)
HALF = 40
GRID = 10


def _exp_body(x_ref, o_ref):
    o_ref[...] = jnp.exp(x_ref[...])


def _log_body(p_ref, o_ref):
    s = p_ref[0] + p_ref[1]
    o_ref[...] = jnp.log(jnp.maximum(s, 1e-30))


def _sc_body(ev, meta, pf, zeros, out, meta_v, p_v, rows_v, s_sh, gsem, msem):
    cid = lax.axis_index("c")
    sid = lax.axis_index("s")
    wid = cid * NS + sid

    pltpu.async_copy(meta.at[pl.ds(wid * BPW, HALF)], meta_v, msem)
    pltpu.async_copy(pf.at[pl.ds(wid * BPW, HALF)], p_v, msem)

    r0 = sid * STRIPE

    @pl.when(sid < NS - 1)
    def _():
        pltpu.sync_copy(zeros.at[pl.ds(r0, STRIPE)],
                        s_sh.at[pl.ds(r0, STRIPE)])

    @pl.when(sid == NS - 1)
    def _():
        pltpu.sync_copy(zeros.at[pl.ds(r0, STRIPE_LAST)],
                        s_sh.at[pl.ds(r0, STRIPE_LAST)])

    pltpu.make_async_copy(meta.at[pl.ds(0, HALF)], meta_v, msem).wait()
    pltpu.make_async_copy(pf.at[pl.ds(0, HALF)], p_v, msem).wait()

    plsc.subcore_barrier()

    for phase in range(2):
        pltpu.async_copy(ev.at[meta_v.at[0, 0]], rows_v.at[0], gsem)

        def outer(t2, carry):
            for u in range(2):
                t = 2 * t2 + u
                b = u

                pltpu.make_async_copy(ev.at[meta_v.at[0, 0]], rows_v.at[b],
                                      gsem).wait()
                t1 = jnp.where(t + 1 < HALF, t + 1, 0)
                pltpu.async_copy(ev.at[meta_v.at[t1, 0]], rows_v.at[1 - b],
                                 gsem)

                def mul_group(g, c):
                    p16 = p_v[t, 0, pl.ds(g * L, L)]
                    for k in range(L):
                        ps = jnp.full((L,), p16[k], jnp.float32)
                        row = g * L + k
                        for j in range(B // L):
                            sl = (b, row, pl.ds(j * L, L))
                            rows_v[sl] = rows_v[sl] * ps
                    return c

                lax.fori_loop(0, BLK // L, mul_group, 0)

                pltpu.sync_copy(rows_v.at[b], s_sh.at[meta_v.at[t, 1]],
                                add=True)
            return carry

        lax.fori_loop(0, HALF // 2, outer, 0)

        pltpu.make_async_copy(ev.at[meta_v.at[0, 0]], rows_v.at[0],
                              gsem).wait()

        if phase == 0:
            pltpu.sync_copy(meta.at[pl.ds(wid * BPW + HALF, HALF)], meta_v)
            pltpu.sync_copy(pf.at[pl.ds(wid * BPW + HALF, HALF)], p_v)

    plsc.subcore_barrier()

    @pl.when(sid < NS - 1)
    def _():
        pltpu.sync_copy(s_sh.at[pl.ds(r0, STRIPE)],
                        out.at[cid, pl.ds(r0, STRIPE)])

    @pl.when(sid == NS - 1)
    def _():
        pltpu.sync_copy(s_sh.at[pl.ds(r0, STRIPE_LAST)],
                        out.at[cid, pl.ds(r0, STRIPE_LAST)])


def kernel(ch_vals, edge_src, edge_dst, params):
    ev = pl.pallas_call(
        _exp_body,
        grid=(GRID,),
        in_specs=[pl.BlockSpec((N // GRID, B), lambda i: (i, 0))],
        out_specs=pl.BlockSpec((N // GRID, B), lambda i: (i, 0)),
        out_shape=jax.ShapeDtypeStruct((N, B), jnp.float32),
    )(ch_vals)

    pad = E_PAD - E
    zpad = jnp.zeros((pad,), jnp.int32)
    src_p = jnp.concatenate([edge_src, zpad]).reshape(BPW, W, BLK)
    dst_p = jnp.concatenate([edge_dst, zpad]).reshape(BPW, W, BLK)
    meta = jnp.stack([src_p, dst_p], axis=2).transpose(1, 0, 2, 3)
    meta = meta.reshape(W * BPW, 2, BLK)
    pf = jnp.concatenate([params, jnp.zeros((pad,), jnp.float32)]
                         ).reshape(BPW, W, 1, BLK).transpose(1, 0, 2, 3)
    pf = pf.reshape(W * BPW, 1, BLK)
    zeros = jnp.zeros((N, B), jnp.float32)

    sc = pl.kernel(
        _sc_body,
        out_type=jax.ShapeDtypeStruct((NC, N, B), jnp.float32),
        mesh=plsc.VectorSubcoreMesh(core_axis_name="c", subcore_axis_name="s"),
        scratch_types=[
            pltpu.VMEM((HALF, 2, BLK), jnp.int32),
            pltpu.VMEM((HALF, 1, BLK), jnp.float32),
            pltpu.VMEM((2, BLK, B), jnp.float32),
            pltpu.VMEM_SHARED((N, B), jnp.float32),
            pltpu.SemaphoreType.DMA,
            pltpu.SemaphoreType.DMA,
        ],
    )
    partial = sc(ev, meta, pf, zeros)

    out = pl.pallas_call(
        _log_body,
        grid=(GRID,),
        in_specs=[pl.BlockSpec((NC, N // GRID, B), lambda i: (0, i, 0))],
        out_specs=pl.BlockSpec((N // GRID, B), lambda i: (i, 0)),
        out_shape=jax.ShapeDtypeStruct((N, B), jnp.float32),
    )(partial)
    return out
